# Initial kernel scaffold; baseline (speedup 1.0000x reference)
import jax
import jax.numpy as jnp
from jax.experimental import pallas as pl


def kernel(input_ids, table):
    raise NotImplementedError("write your pallas kernel here")


# speedup vs baseline: 3.4655x; 3.4655x over previous
import functools
import math

import jax
import jax.numpy as jnp
from jax import lax
from jax.experimental import pallas as pl
from jax.experimental.pallas import tpu as pltpu
from jax.experimental.pallas import tpu_sc as plsc

VOCAB = 100000
D = 64
MAX_LEN = 1024
LANES = 16
NC, NS = 2, 16
NW = NC * NS
ROWS = 1024 * 1024
ROWS_PER_W = ROWS // NW
CHUNK = 256
N_CHUNKS = ROWS_PER_W // CHUNK
IDX_ROWS = CHUNK // 128


def _sin_pe(max_len, d_model):
    pos = jnp.arange(0, max_len, dtype=jnp.float32)[:, None]
    div = jnp.exp(jnp.arange(0, d_model, 2, dtype=jnp.float32)
                  * (-(math.log(10000.0) / d_model)))
    pe = jnp.zeros((max_len, d_model), dtype=jnp.float32)
    pe = pe.at[:, 0::2].set(jnp.sin(pos * div))
    pe = pe.at[:, 1::2].set(jnp.cos(pos * div))
    return pe


GROUP = 4
N_GROUPS = N_CHUNKS // GROUP


def _sc_body(table_hbm, ids_hbm, pe_hbm, out_hbm, idx_v, rows_v, pe_v,
             gsem, osem):
    wid = lax.axis_index("s") * NC + lax.axis_index("c")
    pltpu.sync_copy(pe_hbm, pe_v)

    def group_body(g, carry):
        rb = wid * (ROWS_PER_W // 128) + g * (GROUP * IDX_ROWS)
        pltpu.sync_copy(ids_hbm.at[pl.ds(rb, GROUP * IDX_ROWS)], idx_v)
        for u in range(GROUP):
            base = wid * ROWS_PER_W + (g * GROUP + u) * CHUNK
            cps = [
                pltpu.async_copy(table_hbm.at[idx_v.at[u * IDX_ROWS + j]],
                                 rows_v.at[pl.ds(j * 128, 128)], gsem)
                for j in range(IDX_ROWS)
            ]
            for cp in cps:
                cp.wait()
            p0 = u * CHUNK

            def row_body(i, carry2):
                for j in range(D // LANES):
                    v = pe_v[p0 + i, pl.ds(j * LANES, LANES)]
                    plsc.addupdate(rows_v.at[i, pl.ds(j * LANES, LANES)], v)
                return carry2

            lax.fori_loop(0, CHUNK, row_body, 0, unroll=2)
            pltpu.async_copy(rows_v, out_hbm.at[pl.ds(base, CHUNK)],
                             osem).wait()
        return carry

    lax.fori_loop(0, N_GROUPS, group_body, 0)


@jax.jit
def _tpe_sc(ids_flat2d, table, pe):
    mesh = plsc.VectorSubcoreMesh(core_axis_name="c", subcore_axis_name="s")
    k = functools.partial(
        pl.kernel,
        out_type=jax.ShapeDtypeStruct((ROWS, D), jnp.float32),
        mesh=mesh,
        scratch_types=[
            pltpu.VMEM((GROUP * IDX_ROWS, 128), jnp.int32),
            pltpu.VMEM((CHUNK, D), jnp.float32),
            pltpu.VMEM((MAX_LEN, D), jnp.float32),
            pltpu.SemaphoreType.DMA,
            pltpu.SemaphoreType.DMA,
        ],
        compiler_params=pltpu.CompilerParams(use_tc_tiling_on_sc=False),
    )(_sc_body)
    return k(table, ids_flat2d, pe)


def kernel(input_ids, table):
    b, s = input_ids.shape
    ids = input_ids.reshape(ROWS // 128, 128).astype(jnp.int32)
    pe = _sin_pe(MAX_LEN, D)
    out = _tpe_sc(ids, table, pe)
    return out.reshape(b, s, D)
